# Initial kernel scaffold; baseline (speedup 1.0000x reference)
import jax
import jax.numpy as jnp
from jax.experimental import pallas as pl


def kernel(x, edge_index, Wl1, Wr1, b1, Wl2, Wr2, b2):
    raise NotImplementedError("write your pallas kernel here")


# speedup vs baseline: 11.9898x; 11.9898x over previous
import functools

import jax
import jax.numpy as jnp
from jax import lax
from jax.experimental import pallas as pl
from jax.experimental.pallas import tpu as pltpu
from jax.experimental.pallas import tpu_sc as plsc

N = 10000
E = 320000
D = 128

NP = 10240
NC = 2
NS = 16
NW = NC * NS
SLAB = NP // NS
CHUNK = 128
EPW = 10112
NCH = EPW // CHUNK
EP = NW * EPW

_mesh = plsc.VectorSubcoreMesh(core_axis_name="c", subcore_axis_name="s")


@functools.partial(
    pl.kernel,
    out_type=[
        jax.ShapeDtypeStruct((NC * NP, D), jnp.float32),
        jax.ShapeDtypeStruct((NC * NP,), jnp.float32),
    ],
    mesh=_mesh,
    scratch_types=[
        pltpu.VMEM((NCH, CHUNK), jnp.int32),
        pltpu.VMEM((NCH, CHUNK), jnp.int32),
        pltpu.VMEM((CHUNK, D), jnp.float32),
        pltpu.VMEM((CHUNK,), jnp.float32),
        pltpu.VMEM_SHARED((NP, D), jnp.float32),
        pltpu.VMEM_SHARED((NP,), jnp.float32),
        pltpu.SemaphoreType.DMA,
    ],
)
def _sc_agg_rows(src_hbm, dst_hbm, x_hbm, z2_hbm, z1_hbm,
                 agg_out, deg_out, srcv, dstv, rows, ones, acc_a, acc_d, sem):
    cid = lax.axis_index("c")
    sid = lax.axis_index("s")
    wid = cid * NS + sid

    pltpu.sync_copy(z2_hbm, acc_a.at[pl.ds(sid * SLAB, SLAB)])
    pltpu.sync_copy(z1_hbm.at[pl.ds(0, SLAB)], acc_d.at[pl.ds(sid * SLAB, SLAB)])
    pltpu.sync_copy(src_hbm.at[wid], srcv)
    pltpu.sync_copy(dst_hbm.at[wid], dstv)
    for g in range(CHUNK // 16):
        ones[pl.ds(g * 16, 16)] = jnp.ones((16,), jnp.float32)
    plsc.subcore_barrier()

    def body(j, carry):
        pltpu.async_copy(x_hbm.at[srcv.at[j]], rows, sem).wait()
        pltpu.sync_copy(rows, acc_a.at[dstv.at[j]], add=True)
        pltpu.sync_copy(ones, acc_d.at[dstv.at[j]], add=True)
        return carry

    lax.fori_loop(0, NCH, body, 0)
    plsc.subcore_barrier()

    base = cid * NP + sid * SLAB
    pltpu.sync_copy(acc_a.at[pl.ds(sid * SLAB, SLAB)], agg_out.at[pl.ds(base, SLAB)])
    pltpu.sync_copy(acc_d.at[pl.ds(sid * SLAB, SLAB)], deg_out.at[pl.ds(base, SLAB)])


@functools.partial(
    pl.kernel,
    out_type=jax.ShapeDtypeStruct((NC * NP,), jnp.float32),
    mesh=_mesh,
    scratch_types=[
        pltpu.VMEM((NCH, CHUNK), jnp.int32),
        pltpu.VMEM((NCH, CHUNK), jnp.int32),
        pltpu.VMEM((CHUNK,), jnp.float32),
        pltpu.VMEM_SHARED((NP,), jnp.float32),
        pltpu.VMEM_SHARED((NP,), jnp.float32),
        pltpu.SemaphoreType.DMA,
    ],
)
def _sc_agg_scalar(src_hbm, dst_hbm, p_hbm, z1_hbm,
                   aggp_out, srcv, dstv, vals, p_sh, acc_p, sem):
    cid = lax.axis_index("c")
    sid = lax.axis_index("s")
    wid = cid * NS + sid

    pltpu.sync_copy(z1_hbm.at[pl.ds(0, SLAB)], acc_p.at[pl.ds(sid * SLAB, SLAB)])
    pltpu.sync_copy(p_hbm.at[pl.ds(sid * SLAB, SLAB)], p_sh.at[pl.ds(sid * SLAB, SLAB)])
    pltpu.sync_copy(src_hbm.at[wid], srcv)
    pltpu.sync_copy(dst_hbm.at[wid], dstv)
    plsc.subcore_barrier()

    def body(j, carry):
        pltpu.async_copy(p_sh.at[srcv.at[j]], vals, sem).wait()
        pltpu.sync_copy(vals, acc_p.at[dstv.at[j]], add=True)
        return carry

    lax.fori_loop(0, NCH, body, 0)
    plsc.subcore_barrier()

    base = cid * NP + sid * SLAB
    pltpu.sync_copy(acc_p.at[pl.ds(sid * SLAB, SLAB)], aggp_out.at[pl.ds(base, SLAB)])


BR = 1000


def _tc_layer1_body(a0, a1, d0, d1, x, wl1, wr1, b1, wl2, wr2,
                    p_ref, q_ref, degc_ref):
    degc = jnp.maximum(d0[...] + d1[...], 1.0)
    mean = (a0[...] + a1[...]) / degc
    h = jnp.maximum(
        jnp.dot(mean, wl1[...], preferred_element_type=jnp.float32)
        + jnp.dot(x[...], wr1[...], preferred_element_type=jnp.float32)
        + b1[...],
        0.0,
    )
    p_ref[...] = jnp.sum(h * wl2[...], axis=1, keepdims=True)
    q_ref[...] = jnp.sum(h * wr2[...], axis=1, keepdims=True)
    degc_ref[...] = degc


_tc_layer1 = pl.pallas_call(
    _tc_layer1_body,
    grid=(N // BR,),
    in_specs=[
        pl.BlockSpec((BR, D), lambda i: (i, 0)),
        pl.BlockSpec((BR, D), lambda i: (i, 0)),
        pl.BlockSpec((BR, 1), lambda i: (i, 0)),
        pl.BlockSpec((BR, 1), lambda i: (i, 0)),
        pl.BlockSpec((BR, D), lambda i: (i, 0)),
        pl.BlockSpec((D, D), lambda i: (0, 0)),
        pl.BlockSpec((D, D), lambda i: (0, 0)),
        pl.BlockSpec((1, D), lambda i: (0, 0)),
        pl.BlockSpec((1, D), lambda i: (0, 0)),
        pl.BlockSpec((1, D), lambda i: (0, 0)),
    ],
    out_specs=[
        pl.BlockSpec((BR, 1), lambda i: (i, 0)),
        pl.BlockSpec((BR, 1), lambda i: (i, 0)),
        pl.BlockSpec((BR, 1), lambda i: (i, 0)),
    ],
    out_shape=[
        jax.ShapeDtypeStruct((N, 1), jnp.float32),
        jax.ShapeDtypeStruct((N, 1), jnp.float32),
        jax.ShapeDtypeStruct((N, 1), jnp.float32),
    ],
)


def _tc_final_body(ap0, ap1, degc, q, b2, out_ref):
    out_ref[...] = (ap0[...] + ap1[...]) / degc[...] + q[...] + b2[...]


_tc_final = pl.pallas_call(
    _tc_final_body,
    grid=(N // BR,),
    in_specs=[
        pl.BlockSpec((BR, 1), lambda i: (i, 0)),
        pl.BlockSpec((BR, 1), lambda i: (i, 0)),
        pl.BlockSpec((BR, 1), lambda i: (i, 0)),
        pl.BlockSpec((BR, 1), lambda i: (i, 0)),
        pl.BlockSpec((1, 1), lambda i: (0, 0)),
    ],
    out_specs=pl.BlockSpec((BR, 1), lambda i: (i, 0)),
    out_shape=jax.ShapeDtypeStruct((N, 1), jnp.float32),
)


def kernel(x, edge_index, Wl1, Wr1, b1, Wl2, Wr2, b2):
    src = edge_index[0]
    dst = edge_index[1]
    npad = EP - E
    ar = jnp.arange(npad, dtype=jnp.int32)
    pad_idx = (N + (ar % (NP - N))).astype(jnp.int32)
    srcp = jnp.concatenate([src, pad_idx]).reshape(NW, NCH, CHUNK)
    dstp = jnp.concatenate([dst, pad_idx]).reshape(NW, NCH, CHUNK)
    x_pad = jnp.concatenate([x, jnp.zeros((NP - N, D), jnp.float32)], axis=0)
    z2 = jnp.zeros((SLAB, D), jnp.float32)
    z1 = jnp.zeros((NP,), jnp.float32)

    agg, deg = _sc_agg_rows(srcp, dstp, x_pad, z2, z1)
    p, q, degc = _tc_layer1(
        agg[:N], agg[NP:NP + N], deg[:N, None], deg[NP:NP + N, None],
        x, Wl1, Wr1, b1[None, :], Wl2.T, Wr2.T,
    )
    p_pad = jnp.concatenate([p[:, 0], jnp.zeros((NP - N,), jnp.float32)])
    aggp = _sc_agg_scalar(srcp, dstp, p_pad, z1)
    return _tc_final(aggp[:N, None], aggp[NP:NP + N, None], degc, q, b2[None, :])


# speedup vs baseline: 15.1409x; 1.2628x over previous
import functools

import jax
import jax.numpy as jnp
from jax import lax
from jax.experimental import pallas as pl
from jax.experimental.pallas import tpu as pltpu
from jax.experimental.pallas import tpu_sc as plsc

N = 10000
E = 320000
D = 128

NP = 10240
NC = 2
NS = 16
NW = NC * NS
SLAB = NP // NS
CHUNK = 128
EPW = 10240
NCH = EPW // CHUNK
EP = NW * EPW

_mesh = plsc.VectorSubcoreMesh(core_axis_name="c", subcore_axis_name="s")


@functools.partial(
    pl.kernel,
    out_type=[
        jax.ShapeDtypeStruct((NC * NP, D), jnp.float32),
        jax.ShapeDtypeStruct((NC * NP,), jnp.float32),
    ],
    mesh=_mesh,
    scratch_types=[
        pltpu.VMEM((CHUNK,), jnp.int32),
        pltpu.VMEM((CHUNK,), jnp.int32),
        pltpu.VMEM((NCH, CHUNK), jnp.int32),
        pltpu.VMEM((CHUNK, D), jnp.float32),
        pltpu.VMEM((CHUNK, D), jnp.float32),
        pltpu.VMEM((CHUNK,), jnp.float32),
        pltpu.VMEM_SHARED((NP, D), jnp.float32),
        pltpu.VMEM_SHARED((NP,), jnp.float32),
        pltpu.SemaphoreType.DMA,
        pltpu.SemaphoreType.DMA,
    ],
)
def _sc_agg_rows(src_hbm, dst_hbm, x_hbm, z2_hbm, z1_hbm,
                 agg_out, deg_out, srcb0, srcb1, dstv, rows0, rows1, ones,
                 acc_a, acc_d, sem0, sem1):
    cid = lax.axis_index("c")
    sid = lax.axis_index("s")
    wid = cid * NS + sid

    pltpu.sync_copy(z2_hbm, acc_a.at[pl.ds(sid * SLAB, SLAB)])
    pltpu.sync_copy(z1_hbm.at[pl.ds(0, SLAB)], acc_d.at[pl.ds(sid * SLAB, SLAB)])
    pltpu.sync_copy(dst_hbm.at[wid], dstv)
    for g in range(CHUNK // 16):
        ones[pl.ds(g * 16, 16)] = jnp.ones((16,), jnp.float32)
    plsc.subcore_barrier()

    pltpu.sync_copy(src_hbm.at[wid, pl.ds(0, CHUNK)], srcb0)
    pltpu.async_copy(x_hbm.at[srcb0], rows0, sem0)
    pltpu.sync_copy(src_hbm.at[wid, pl.ds(CHUNK, CHUNK)], srcb1)

    def body(t, carry):
        j = 2 * t
        pltpu.make_async_copy(x_hbm.at[srcb0], rows0, sem0).wait()
        pltpu.async_copy(x_hbm.at[srcb1], rows1, sem1)

        @pl.when(j + 2 < NCH)
        def _():
            pltpu.sync_copy(src_hbm.at[wid, pl.ds((j + 2) * CHUNK, CHUNK)], srcb0)

        pltpu.sync_copy(rows0, acc_a.at[dstv.at[j]], add=True)
        pltpu.sync_copy(ones, acc_d.at[dstv.at[j]], add=True)
        pltpu.make_async_copy(x_hbm.at[srcb1], rows1, sem1).wait()

        @pl.when(j + 2 < NCH)
        def _():
            pltpu.async_copy(x_hbm.at[srcb0], rows0, sem0)

        @pl.when(j + 3 < NCH)
        def _():
            pltpu.sync_copy(src_hbm.at[wid, pl.ds((j + 3) * CHUNK, CHUNK)], srcb1)

        pltpu.sync_copy(rows1, acc_a.at[dstv.at[j + 1]], add=True)
        pltpu.sync_copy(ones, acc_d.at[dstv.at[j + 1]], add=True)
        return carry

    lax.fori_loop(0, NCH // 2, body, 0)
    plsc.subcore_barrier()

    base = cid * NP + sid * SLAB
    pltpu.sync_copy(acc_a.at[pl.ds(sid * SLAB, SLAB)], agg_out.at[pl.ds(base, SLAB)])
    pltpu.sync_copy(acc_d.at[pl.ds(sid * SLAB, SLAB)], deg_out.at[pl.ds(base, SLAB)])


@functools.partial(
    pl.kernel,
    out_type=jax.ShapeDtypeStruct((NC * NP,), jnp.float32),
    mesh=_mesh,
    scratch_types=[
        pltpu.VMEM((NCH, CHUNK), jnp.int32),
        pltpu.VMEM((NCH, CHUNK), jnp.int32),
        pltpu.VMEM((CHUNK,), jnp.float32),
        pltpu.VMEM((CHUNK,), jnp.float32),
        pltpu.VMEM_SHARED((NP,), jnp.float32),
        pltpu.VMEM_SHARED((NP,), jnp.float32),
        pltpu.SemaphoreType.DMA,
        pltpu.SemaphoreType.DMA,
    ],
)
def _sc_agg_scalar(src_hbm, dst_hbm, p_hbm, z1_hbm,
                   aggp_out, srcv, dstv, vals0, vals1, p_sh, acc_p, sem0, sem1):
    cid = lax.axis_index("c")
    sid = lax.axis_index("s")
    wid = cid * NS + sid

    pltpu.sync_copy(z1_hbm.at[pl.ds(0, SLAB)], acc_p.at[pl.ds(sid * SLAB, SLAB)])
    pltpu.sync_copy(p_hbm.at[pl.ds(sid * SLAB, SLAB)], p_sh.at[pl.ds(sid * SLAB, SLAB)])
    pltpu.sync_copy(src_hbm.at[wid], srcv)
    pltpu.sync_copy(dst_hbm.at[wid], dstv)
    plsc.subcore_barrier()

    pltpu.async_copy(p_sh.at[srcv.at[0]], vals0, sem0)

    def body(t, carry):
        j = 2 * t
        pltpu.make_async_copy(p_sh.at[srcv.at[j]], vals0, sem0).wait()
        pltpu.async_copy(p_sh.at[srcv.at[j + 1]], vals1, sem1)
        pltpu.sync_copy(vals0, acc_p.at[dstv.at[j]], add=True)
        pltpu.make_async_copy(p_sh.at[srcv.at[j + 1]], vals1, sem1).wait()

        @pl.when(t < NCH // 2 - 1)
        def _():
            pltpu.async_copy(p_sh.at[srcv.at[j + 2]], vals0, sem0)

        pltpu.sync_copy(vals1, acc_p.at[dstv.at[j + 1]], add=True)
        return carry

    lax.fori_loop(0, NCH // 2, body, 0)
    plsc.subcore_barrier()

    base = cid * NP + sid * SLAB
    pltpu.sync_copy(acc_p.at[pl.ds(sid * SLAB, SLAB)], aggp_out.at[pl.ds(base, SLAB)])


BR = 1000


def _tc_layer1_body(a0, a1, d0, d1, x, wl1, wr1, b1, wl2, wr2,
                    p_ref, q_ref, degc_ref):
    degc = jnp.maximum(d0[...] + d1[...], 1.0)
    mean = (a0[...] + a1[...]) / degc
    h = jnp.maximum(
        jnp.dot(mean, wl1[...], preferred_element_type=jnp.float32)
        + jnp.dot(x[...], wr1[...], preferred_element_type=jnp.float32)
        + b1[...],
        0.0,
    )
    p_ref[...] = jnp.sum(h * wl2[...], axis=1, keepdims=True)
    q_ref[...] = jnp.sum(h * wr2[...], axis=1, keepdims=True)
    degc_ref[...] = degc


_tc_layer1 = pl.pallas_call(
    _tc_layer1_body,
    grid=(N // BR,),
    in_specs=[
        pl.BlockSpec((BR, D), lambda i: (i, 0)),
        pl.BlockSpec((BR, D), lambda i: (i, 0)),
        pl.BlockSpec((BR, 1), lambda i: (i, 0)),
        pl.BlockSpec((BR, 1), lambda i: (i, 0)),
        pl.BlockSpec((BR, D), lambda i: (i, 0)),
        pl.BlockSpec((D, D), lambda i: (0, 0)),
        pl.BlockSpec((D, D), lambda i: (0, 0)),
        pl.BlockSpec((1, D), lambda i: (0, 0)),
        pl.BlockSpec((1, D), lambda i: (0, 0)),
        pl.BlockSpec((1, D), lambda i: (0, 0)),
    ],
    out_specs=[
        pl.BlockSpec((BR, 1), lambda i: (i, 0)),
        pl.BlockSpec((BR, 1), lambda i: (i, 0)),
        pl.BlockSpec((BR, 1), lambda i: (i, 0)),
    ],
    out_shape=[
        jax.ShapeDtypeStruct((N, 1), jnp.float32),
        jax.ShapeDtypeStruct((N, 1), jnp.float32),
        jax.ShapeDtypeStruct((N, 1), jnp.float32),
    ],
)


def _tc_final_body(ap0, ap1, degc, q, b2, out_ref):
    out_ref[...] = (ap0[...] + ap1[...]) / degc[...] + q[...] + b2[...]


_tc_final = pl.pallas_call(
    _tc_final_body,
    grid=(N // BR,),
    in_specs=[
        pl.BlockSpec((BR, 1), lambda i: (i, 0)),
        pl.BlockSpec((BR, 1), lambda i: (i, 0)),
        pl.BlockSpec((BR, 1), lambda i: (i, 0)),
        pl.BlockSpec((BR, 1), lambda i: (i, 0)),
        pl.BlockSpec((1, 1), lambda i: (0, 0)),
    ],
    out_specs=pl.BlockSpec((BR, 1), lambda i: (i, 0)),
    out_shape=jax.ShapeDtypeStruct((N, 1), jnp.float32),
)


def kernel(x, edge_index, Wl1, Wr1, b1, Wl2, Wr2, b2):
    src = edge_index[0]
    dst = edge_index[1]
    npad = EP - E
    ar = jnp.arange(npad, dtype=jnp.int32)
    pad_src = (ar % jnp.int32(256)).astype(jnp.int32)
    pad_dst = (N + (ar % (NP - N))).astype(jnp.int32)
    srcp = jnp.concatenate([src, pad_src]).reshape(NW, NCH, CHUNK)
    dstp = jnp.concatenate([dst, pad_dst]).reshape(NW, NCH, CHUNK)
    z2 = jnp.zeros((SLAB, D), jnp.float32)
    z1 = jnp.zeros((NP,), jnp.float32)

    agg, deg = _sc_agg_rows(srcp.reshape(NW, EPW), dstp, x, z2, z1)
    p, q, degc = _tc_layer1(
        agg[:N], agg[NP:NP + N], deg[:N, None], deg[NP:NP + N, None],
        x, Wl1, Wr1, b1[None, :], Wl2.T, Wr2.T,
    )
    p_pad = jnp.concatenate([p[:, 0], jnp.zeros((NP - N,), jnp.float32)])
    aggp = _sc_agg_scalar(srcp, dstp, p_pad, z1)
    return _tc_final(aggp[:N, None], aggp[NP:NP + N, None], degc, q, b2[None, :])


# speedup vs baseline: 15.2023x; 1.0041x over previous
import functools

import jax
import jax.numpy as jnp
from jax import lax
from jax.experimental import pallas as pl
from jax.experimental.pallas import tpu as pltpu
from jax.experimental.pallas import tpu_sc as plsc

N = 10000
E = 320000
D = 128

NP = 10240
NC = 2
NS = 16
NW = NC * NS
SLAB = NP // NS
CHUNK = 128
EPW = 10240
NCH = EPW // CHUNK
EP = NW * EPW

_mesh = plsc.VectorSubcoreMesh(core_axis_name="c", subcore_axis_name="s")


@functools.partial(
    pl.kernel,
    out_type=[
        jax.ShapeDtypeStruct((NC * NP, D), jnp.float32),
        jax.ShapeDtypeStruct((NC * NP,), jnp.float32),
    ],
    mesh=_mesh,
    scratch_types=[
        pltpu.VMEM((CHUNK,), jnp.int32),
        pltpu.VMEM((CHUNK,), jnp.int32),
        pltpu.VMEM((NCH, CHUNK), jnp.int32),
        pltpu.VMEM((CHUNK, D), jnp.float32),
        pltpu.VMEM((CHUNK, D), jnp.float32),
        pltpu.VMEM((CHUNK,), jnp.float32),
        pltpu.VMEM_SHARED((NP, D), jnp.float32),
        pltpu.VMEM_SHARED((NP,), jnp.float32),
        pltpu.SemaphoreType.DMA,
        pltpu.SemaphoreType.DMA,
        pltpu.SemaphoreType.DMA,
        pltpu.SemaphoreType.DMA,
        pltpu.SemaphoreType.DMA,
        pltpu.SemaphoreType.DMA,
    ],
)
def _sc_agg_rows(src_hbm, dst_hbm, x_hbm, z2_hbm, z1_hbm,
                 agg_out, deg_out, srcb0, srcb1, dstv, rows0, rows1, ones,
                 acc_a, acc_d, sem0, sem1, semS0, semS1, semD0, semD1):
    cid = lax.axis_index("c")
    sid = lax.axis_index("s")
    wid = cid * NS + sid

    pltpu.sync_copy(z2_hbm, acc_a.at[pl.ds(sid * SLAB, SLAB)])
    pltpu.sync_copy(z1_hbm.at[pl.ds(0, SLAB)], acc_d.at[pl.ds(sid * SLAB, SLAB)])
    pltpu.sync_copy(dst_hbm.at[wid], dstv)
    for g in range(CHUNK // 16):
        ones[pl.ds(g * 16, 16)] = jnp.ones((16,), jnp.float32)
    plsc.subcore_barrier()

    H = CHUNK // 2

    def _gather(srcb, rows):
        pltpu.async_copy(x_hbm.at[srcb.at[pl.ds(0, H)]], rows.at[pl.ds(0, H)], sem0)
        pltpu.async_copy(x_hbm.at[srcb.at[pl.ds(H, H)]], rows.at[pl.ds(H, H)], sem1)

    def _gwait(srcb, rows):
        pltpu.make_async_copy(x_hbm.at[srcb.at[pl.ds(0, H)]], rows.at[pl.ds(0, H)], sem0).wait()
        pltpu.make_async_copy(x_hbm.at[srcb.at[pl.ds(H, H)]], rows.at[pl.ds(H, H)], sem1).wait()

    pltpu.sync_copy(src_hbm.at[wid, pl.ds(0, CHUNK)], srcb0)
    _gather(srcb0, rows0)
    pltpu.sync_copy(src_hbm.at[wid, pl.ds(CHUNK, CHUNK)], srcb1)

    def body(t, carry):
        j = 2 * t
        _gwait(srcb0, rows0)
        pltpu.async_copy(rows0, acc_a.at[dstv.at[j]], semS0, add=True)
        pltpu.async_copy(ones, acc_d.at[dstv.at[j]], semD0, add=True)

        @pl.when(t > 0)
        def _():
            pltpu.make_async_copy(rows1, acc_a.at[dstv.at[j]], semS1).wait()
            pltpu.make_async_copy(ones, acc_d.at[dstv.at[j]], semD1).wait()

        _gather(srcb1, rows1)

        @pl.when(j + 2 < NCH)
        def _():
            pltpu.sync_copy(src_hbm.at[wid, pl.ds((j + 2) * CHUNK, CHUNK)], srcb0)

        _gwait(srcb1, rows1)
        pltpu.async_copy(rows1, acc_a.at[dstv.at[j + 1]], semS1, add=True)
        pltpu.async_copy(ones, acc_d.at[dstv.at[j + 1]], semD1, add=True)

        pltpu.make_async_copy(rows0, acc_a.at[dstv.at[j]], semS0).wait()
        pltpu.make_async_copy(ones, acc_d.at[dstv.at[j]], semD0).wait()

        @pl.when(j + 2 < NCH)
        def _():
            _gather(srcb0, rows0)

        @pl.when(j + 3 < NCH)
        def _():
            pltpu.sync_copy(src_hbm.at[wid, pl.ds((j + 3) * CHUNK, CHUNK)], srcb1)

        return carry

    lax.fori_loop(0, NCH // 2, body, 0)
    pltpu.make_async_copy(rows1, acc_a.at[dstv.at[NCH - 1]], semS1).wait()
    pltpu.make_async_copy(ones, acc_d.at[dstv.at[NCH - 1]], semD1).wait()
    plsc.subcore_barrier()

    base = cid * NP + sid * SLAB
    pltpu.sync_copy(acc_a.at[pl.ds(sid * SLAB, SLAB)], agg_out.at[pl.ds(base, SLAB)])
    pltpu.sync_copy(acc_d.at[pl.ds(sid * SLAB, SLAB)], deg_out.at[pl.ds(base, SLAB)])


@functools.partial(
    pl.kernel,
    out_type=jax.ShapeDtypeStruct((NC * NP,), jnp.float32),
    mesh=_mesh,
    scratch_types=[
        pltpu.VMEM((NCH, CHUNK), jnp.int32),
        pltpu.VMEM((NCH, CHUNK), jnp.int32),
        pltpu.VMEM((CHUNK,), jnp.float32),
        pltpu.VMEM((CHUNK,), jnp.float32),
        pltpu.VMEM_SHARED((NP,), jnp.float32),
        pltpu.VMEM_SHARED((NP,), jnp.float32),
        pltpu.SemaphoreType.DMA,
        pltpu.SemaphoreType.DMA,
        pltpu.SemaphoreType.DMA,
        pltpu.SemaphoreType.DMA,
    ],
)
def _sc_agg_scalar(src_hbm, dst_hbm, p_hbm, z1_hbm,
                   aggp_out, srcv, dstv, vals0, vals1, p_sh, acc_p,
                   sem0, sem1, semS0, semS1):
    cid = lax.axis_index("c")
    sid = lax.axis_index("s")
    wid = cid * NS + sid

    pltpu.sync_copy(z1_hbm.at[pl.ds(0, SLAB)], acc_p.at[pl.ds(sid * SLAB, SLAB)])
    pltpu.sync_copy(p_hbm.at[pl.ds(sid * SLAB, SLAB)], p_sh.at[pl.ds(sid * SLAB, SLAB)])
    pltpu.sync_copy(src_hbm.at[wid], srcv)
    pltpu.sync_copy(dst_hbm.at[wid], dstv)
    plsc.subcore_barrier()

    pltpu.async_copy(p_sh.at[srcv.at[0]], vals0, sem0)

    def body(t, carry):
        j = 2 * t
        pltpu.make_async_copy(p_sh.at[srcv.at[j]], vals0, sem0).wait()
        pltpu.async_copy(vals0, acc_p.at[dstv.at[j]], semS0, add=True)

        @pl.when(t > 0)
        def _():
            pltpu.make_async_copy(vals1, acc_p.at[dstv.at[j]], semS1).wait()

        pltpu.async_copy(p_sh.at[srcv.at[j + 1]], vals1, sem1)
        pltpu.make_async_copy(p_sh.at[srcv.at[j + 1]], vals1, sem1).wait()
        pltpu.async_copy(vals1, acc_p.at[dstv.at[j + 1]], semS1, add=True)
        pltpu.make_async_copy(vals0, acc_p.at[dstv.at[j]], semS0).wait()

        @pl.when(j + 2 < NCH)
        def _():
            pltpu.async_copy(p_sh.at[srcv.at[j + 2]], vals0, sem0)

        return carry

    lax.fori_loop(0, NCH // 2, body, 0)
    pltpu.make_async_copy(vals1, acc_p.at[dstv.at[NCH - 1]], semS1).wait()
    plsc.subcore_barrier()

    base = cid * NP + sid * SLAB
    pltpu.sync_copy(acc_p.at[pl.ds(sid * SLAB, SLAB)], aggp_out.at[pl.ds(base, SLAB)])


BR = 1000


def _tc_layer1_body(a0, a1, d0, d1, x, wl1, wr1, b1, wl2, wr2,
                    p_ref, q_ref, degc_ref):
    degc = jnp.maximum(d0[...] + d1[...], 1.0)
    mean = (a0[...] + a1[...]) / degc
    h = jnp.maximum(
        jnp.dot(mean, wl1[...], preferred_element_type=jnp.float32)
        + jnp.dot(x[...], wr1[...], preferred_element_type=jnp.float32)
        + b1[...],
        0.0,
    )
    p_ref[...] = jnp.sum(h * wl2[...], axis=1, keepdims=True)
    q_ref[...] = jnp.sum(h * wr2[...], axis=1, keepdims=True)
    degc_ref[...] = degc


_tc_layer1 = pl.pallas_call(
    _tc_layer1_body,
    grid=(N // BR,),
    in_specs=[
        pl.BlockSpec((BR, D), lambda i: (i, 0)),
        pl.BlockSpec((BR, D), lambda i: (i, 0)),
        pl.BlockSpec((BR, 1), lambda i: (i, 0)),
        pl.BlockSpec((BR, 1), lambda i: (i, 0)),
        pl.BlockSpec((BR, D), lambda i: (i, 0)),
        pl.BlockSpec((D, D), lambda i: (0, 0)),
        pl.BlockSpec((D, D), lambda i: (0, 0)),
        pl.BlockSpec((1, D), lambda i: (0, 0)),
        pl.BlockSpec((1, D), lambda i: (0, 0)),
        pl.BlockSpec((1, D), lambda i: (0, 0)),
    ],
    out_specs=[
        pl.BlockSpec((BR, 1), lambda i: (i, 0)),
        pl.BlockSpec((BR, 1), lambda i: (i, 0)),
        pl.BlockSpec((BR, 1), lambda i: (i, 0)),
    ],
    out_shape=[
        jax.ShapeDtypeStruct((N, 1), jnp.float32),
        jax.ShapeDtypeStruct((N, 1), jnp.float32),
        jax.ShapeDtypeStruct((N, 1), jnp.float32),
    ],
)


def _tc_final_body(ap0, ap1, degc, q, b2, out_ref):
    out_ref[...] = (ap0[...] + ap1[...]) / degc[...] + q[...] + b2[...]


_tc_final = pl.pallas_call(
    _tc_final_body,
    grid=(N // BR,),
    in_specs=[
        pl.BlockSpec((BR, 1), lambda i: (i, 0)),
        pl.BlockSpec((BR, 1), lambda i: (i, 0)),
        pl.BlockSpec((BR, 1), lambda i: (i, 0)),
        pl.BlockSpec((BR, 1), lambda i: (i, 0)),
        pl.BlockSpec((1, 1), lambda i: (0, 0)),
    ],
    out_specs=pl.BlockSpec((BR, 1), lambda i: (i, 0)),
    out_shape=jax.ShapeDtypeStruct((N, 1), jnp.float32),
)


def kernel(x, edge_index, Wl1, Wr1, b1, Wl2, Wr2, b2):
    src = edge_index[0]
    dst = edge_index[1]
    npad = EP - E
    ar = jnp.arange(npad, dtype=jnp.int32)
    pad_src = (ar % jnp.int32(256)).astype(jnp.int32)
    pad_dst = (N + (ar % (NP - N))).astype(jnp.int32)
    srcp = jnp.concatenate([src, pad_src]).reshape(NW, NCH, CHUNK)
    dstp = jnp.concatenate([dst, pad_dst]).reshape(NW, NCH, CHUNK)
    z2 = jnp.zeros((SLAB, D), jnp.float32)
    z1 = jnp.zeros((NP,), jnp.float32)

    agg, deg = _sc_agg_rows(srcp.reshape(NW, EPW), dstp, x, z2, z1)
    p, q, degc = _tc_layer1(
        agg[:N], agg[NP:NP + N], deg[:N, None], deg[NP:NP + N, None],
        x, Wl1, Wr1, b1[None, :], Wl2.T, Wr2.T,
    )
    p_pad = jnp.concatenate([p[:, 0], jnp.zeros((NP - N,), jnp.float32)])
    aggp = _sc_agg_scalar(srcp, dstp, p_pad, z1)
    return _tc_final(aggp[:N, None], aggp[NP:NP + N, None], degc, q, b2[None, :])


# speedup vs baseline: 15.8660x; 1.0437x over previous
import functools

import jax
import jax.numpy as jnp
from jax import lax
from jax.experimental import pallas as pl
from jax.experimental.pallas import tpu as pltpu
from jax.experimental.pallas import tpu_sc as plsc

N = 10000
E = 320000
D = 128

NP = 10240
NC = 2
NS = 16
NW = NC * NS
SLAB = NP // NS
CHUNK = 128
EPW = 10240
NCH = EPW // CHUNK
EP = NW * EPW

_mesh = plsc.VectorSubcoreMesh(core_axis_name="c", subcore_axis_name="s")


@functools.partial(
    pl.kernel,
    out_type=[
        jax.ShapeDtypeStruct((NC * NP, D), jnp.float32),
        jax.ShapeDtypeStruct((NC * NP,), jnp.float32),
    ],
    mesh=_mesh,
    scratch_types=[
        pltpu.VMEM((CHUNK,), jnp.int32),
        pltpu.VMEM((CHUNK,), jnp.int32),
        pltpu.VMEM((NCH, CHUNK), jnp.int32),
        pltpu.VMEM((CHUNK, D), jnp.float32),
        pltpu.VMEM((CHUNK, D), jnp.float32),
        pltpu.VMEM((CHUNK,), jnp.float32),
        pltpu.VMEM_SHARED((NP, D), jnp.float32),
        pltpu.VMEM_SHARED((NP,), jnp.float32),
        pltpu.SemaphoreType.DMA,
        pltpu.SemaphoreType.DMA,
        pltpu.SemaphoreType.DMA,
        pltpu.SemaphoreType.DMA,
        pltpu.SemaphoreType.DMA,
        pltpu.SemaphoreType.DMA,
    ],
)
def _sc_agg_rows(src_hbm, dst_hbm, x_hbm, z2_hbm, z1_hbm,
                 agg_out, deg_out, srcb0, srcb1, dstv, rows0, rows1, ones,
                 acc_a, acc_d, sem0, sem1, semS0, semS1, semD0, semD1):
    cid = lax.axis_index("c")
    sid = lax.axis_index("s")
    wid = cid * NS + sid

    pltpu.sync_copy(z2_hbm, acc_a.at[pl.ds(sid * SLAB, SLAB)])
    pltpu.sync_copy(z1_hbm.at[pl.ds(0, SLAB)], acc_d.at[pl.ds(sid * SLAB, SLAB)])
    pltpu.sync_copy(dst_hbm.at[wid], dstv)
    for g in range(CHUNK // 16):
        ones[pl.ds(g * 16, 16)] = jnp.ones((16,), jnp.float32)
    plsc.subcore_barrier()

    H = CHUNK // 2

    def _gather(srcb, rows):
        pltpu.async_copy(x_hbm.at[srcb.at[pl.ds(0, H)]], rows.at[pl.ds(0, H)], sem0)
        pltpu.async_copy(x_hbm.at[srcb.at[pl.ds(H, H)]], rows.at[pl.ds(H, H)], sem1)

    def _gwait(srcb, rows):
        pltpu.make_async_copy(x_hbm.at[srcb.at[pl.ds(0, H)]], rows.at[pl.ds(0, H)], sem0).wait()
        pltpu.make_async_copy(x_hbm.at[srcb.at[pl.ds(H, H)]], rows.at[pl.ds(H, H)], sem1).wait()

    pltpu.sync_copy(src_hbm.at[wid, pl.ds(0, CHUNK)], srcb0)
    _gather(srcb0, rows0)
    pltpu.sync_copy(src_hbm.at[wid, pl.ds(CHUNK, CHUNK)], srcb1)

    def body(t, carry):
        j = 2 * t
        _gwait(srcb0, rows0)
        pltpu.async_copy(rows0, acc_a.at[dstv.at[j]], semS0, add=True)
        pltpu.async_copy(ones, acc_d.at[dstv.at[j]], semD0, add=True)

        @pl.when(t > 0)
        def _():
            pltpu.make_async_copy(rows1, acc_a.at[dstv.at[j]], semS1).wait()
            pltpu.make_async_copy(ones, acc_d.at[dstv.at[j]], semD1).wait()

        _gather(srcb1, rows1)

        @pl.when(j + 2 < NCH)
        def _():
            pltpu.sync_copy(src_hbm.at[wid, pl.ds((j + 2) * CHUNK, CHUNK)], srcb0)

        _gwait(srcb1, rows1)
        pltpu.async_copy(rows1, acc_a.at[dstv.at[j + 1]], semS1, add=True)
        pltpu.async_copy(ones, acc_d.at[dstv.at[j + 1]], semD1, add=True)

        pltpu.make_async_copy(rows0, acc_a.at[dstv.at[j]], semS0).wait()
        pltpu.make_async_copy(ones, acc_d.at[dstv.at[j]], semD0).wait()

        @pl.when(j + 2 < NCH)
        def _():
            _gather(srcb0, rows0)

        @pl.when(j + 3 < NCH)
        def _():
            pltpu.sync_copy(src_hbm.at[wid, pl.ds((j + 3) * CHUNK, CHUNK)], srcb1)

        return carry

    lax.fori_loop(0, NCH // 2, body, 0)
    pltpu.make_async_copy(rows1, acc_a.at[dstv.at[NCH - 1]], semS1).wait()
    pltpu.make_async_copy(ones, acc_d.at[dstv.at[NCH - 1]], semD1).wait()
    plsc.subcore_barrier()

    base = cid * NP + sid * SLAB
    pltpu.sync_copy(acc_a.at[pl.ds(sid * SLAB, SLAB)], agg_out.at[pl.ds(base, SLAB)])
    pltpu.sync_copy(acc_d.at[pl.ds(sid * SLAB, SLAB)], deg_out.at[pl.ds(base, SLAB)])


@functools.partial(
    pl.kernel,
    out_type=jax.ShapeDtypeStruct((NC * NP,), jnp.float32),
    mesh=_mesh,
    scratch_types=[
        pltpu.VMEM((NCH, CHUNK), jnp.int32),
        pltpu.VMEM((NCH, CHUNK), jnp.int32),
        pltpu.VMEM((CHUNK,), jnp.float32),
        pltpu.VMEM((CHUNK,), jnp.float32),
        pltpu.VMEM_SHARED((NP,), jnp.float32),
        pltpu.VMEM_SHARED((NP,), jnp.float32),
        pltpu.SemaphoreType.DMA,
        pltpu.SemaphoreType.DMA,
        pltpu.SemaphoreType.DMA,
        pltpu.SemaphoreType.DMA,
    ],
)
def _sc_agg_scalar(src_hbm, dst_hbm, p_hbm, z1_hbm,
                   aggp_out, srcv, dstv, vals0, vals1, p_sh, acc_p,
                   sem0, sem1, semS0, semS1):
    cid = lax.axis_index("c")
    sid = lax.axis_index("s")
    wid = cid * NS + sid

    pltpu.sync_copy(z1_hbm.at[pl.ds(0, SLAB)], acc_p.at[pl.ds(sid * SLAB, SLAB)])
    pltpu.sync_copy(p_hbm.at[pl.ds(sid * SLAB, SLAB)], p_sh.at[pl.ds(sid * SLAB, SLAB)])
    pltpu.sync_copy(src_hbm.at[wid], srcv)
    pltpu.sync_copy(dst_hbm.at[wid], dstv)
    plsc.subcore_barrier()

    pltpu.async_copy(p_sh.at[srcv.at[0]], vals0, sem0)

    def body(t, carry):
        j = 2 * t
        pltpu.make_async_copy(p_sh.at[srcv.at[j]], vals0, sem0).wait()
        pltpu.async_copy(vals0, acc_p.at[dstv.at[j]], semS0, add=True)

        @pl.when(t > 0)
        def _():
            pltpu.make_async_copy(vals1, acc_p.at[dstv.at[j]], semS1).wait()

        pltpu.async_copy(p_sh.at[srcv.at[j + 1]], vals1, sem1)
        pltpu.make_async_copy(p_sh.at[srcv.at[j + 1]], vals1, sem1).wait()
        pltpu.async_copy(vals1, acc_p.at[dstv.at[j + 1]], semS1, add=True)
        pltpu.make_async_copy(vals0, acc_p.at[dstv.at[j]], semS0).wait()

        @pl.when(j + 2 < NCH)
        def _():
            pltpu.async_copy(p_sh.at[srcv.at[j + 2]], vals0, sem0)

        return carry

    lax.fori_loop(0, NCH // 2, body, 0)
    pltpu.make_async_copy(vals1, acc_p.at[dstv.at[NCH - 1]], semS1).wait()
    plsc.subcore_barrier()

    base = cid * NP + sid * SLAB
    pltpu.sync_copy(acc_p.at[pl.ds(sid * SLAB, SLAB)], aggp_out.at[pl.ds(base, SLAB)])


BR = 640


def _tc_layer1_body(a0, a1, d0, d1, x, wl1, wr1, b1, wl2, wr2,
                    p_ref, q_ref, degc_ref):
    degc = jnp.maximum(d0[...] + d1[...], 1.0)
    mean = (a0[...] + a1[...]) / degc
    h = jnp.maximum(
        jnp.dot(mean, wl1[...], preferred_element_type=jnp.float32)
        + jnp.dot(x[...], wr1[...], preferred_element_type=jnp.float32)
        + b1[...],
        0.0,
    )
    p_ref[...] = jnp.sum(h * wl2[...], axis=1, keepdims=True)
    q_ref[...] = jnp.sum(h * wr2[...], axis=1, keepdims=True)
    degc_ref[...] = degc


_tc_layer1 = pl.pallas_call(
    _tc_layer1_body,
    grid=(NP // BR,),
    in_specs=[
        pl.BlockSpec((BR, D), lambda i: (i, 0)),
        pl.BlockSpec((BR, D), lambda i: (i + 16, 0)),
        pl.BlockSpec((BR, 1), lambda i: (i, 0)),
        pl.BlockSpec((BR, 1), lambda i: (i + 16, 0)),
        pl.BlockSpec((BR, D), lambda i: (i, 0)),
        pl.BlockSpec((D, D), lambda i: (0, 0)),
        pl.BlockSpec((D, D), lambda i: (0, 0)),
        pl.BlockSpec((1, D), lambda i: (0, 0)),
        pl.BlockSpec((1, D), lambda i: (0, 0)),
        pl.BlockSpec((1, D), lambda i: (0, 0)),
    ],
    out_specs=[
        pl.BlockSpec((BR, 1), lambda i: (i, 0)),
        pl.BlockSpec((BR, 1), lambda i: (i, 0)),
        pl.BlockSpec((BR, 1), lambda i: (i, 0)),
    ],
    out_shape=[
        jax.ShapeDtypeStruct((NP, 1), jnp.float32),
        jax.ShapeDtypeStruct((NP, 1), jnp.float32),
        jax.ShapeDtypeStruct((NP, 1), jnp.float32),
    ],
)


def _tc_final_body(ap0, ap1, degc, q, b2, out_ref):
    out_ref[...] = (ap0[...] + ap1[...]) / degc[...] + q[...] + b2[...]


_tc_final = pl.pallas_call(
    _tc_final_body,
    grid=(NP // BR,),
    in_specs=[
        pl.BlockSpec((BR, 1), lambda i: (i, 0)),
        pl.BlockSpec((BR, 1), lambda i: (i + 16, 0)),
        pl.BlockSpec((BR, 1), lambda i: (i, 0)),
        pl.BlockSpec((BR, 1), lambda i: (i, 0)),
        pl.BlockSpec((1, 1), lambda i: (0, 0)),
    ],
    out_specs=pl.BlockSpec((BR, 1), lambda i: (i, 0)),
    out_shape=jax.ShapeDtypeStruct((N, 1), jnp.float32),
)


def kernel(x, edge_index, Wl1, Wr1, b1, Wl2, Wr2, b2):
    src = edge_index[0]
    dst = edge_index[1]
    npad = EP - E
    ar = jnp.arange(npad, dtype=jnp.int32)
    pad_src = (ar % jnp.int32(256)).astype(jnp.int32)
    pad_dst = (N + (ar % (NP - N))).astype(jnp.int32)
    srcp = jnp.concatenate([src, pad_src]).reshape(NW, NCH, CHUNK)
    dstp = jnp.concatenate([dst, pad_dst]).reshape(NW, NCH, CHUNK)
    z2 = jnp.zeros((SLAB, D), jnp.float32)
    z1 = jnp.zeros((NP,), jnp.float32)

    agg, deg = _sc_agg_rows(srcp.reshape(NW, EPW), dstp, x, z2, z1)
    deg2 = deg[:, None]
    p, q, degc = _tc_layer1(
        agg, agg, deg2, deg2, x, Wl1, Wr1, b1[None, :], Wl2.T, Wr2.T,
    )
    aggp = _sc_agg_scalar(srcp, dstp, p.reshape(NP), z1)
    aggp2 = aggp[:, None]
    return _tc_final(aggp2, aggp2, degc, q, b2[None, :])
